# baseline (device time: 100478 ns/iter reference)
import jax
import jax.numpy as jnp
from jax import lax
from jax.experimental import pallas as pl
from jax.experimental.pallas import tpu as pltpu

N_DEV = 4
PIECES = 2


def kernel(A, B):
    m, k = A.shape
    _, n = B.shape
    half = m // 2
    mc = half // N_DEV
    pm = mc // PIECES

    def body(a_ref, b_ref, out_ref,
             a_bf, b_bf,
             rsf_buf, rsr_buf, agf_buf, agr_buf,
             rsf_send, rsf_recv, rsr_send, rsr_recv,
             agf_send, agf_recv, agr_send, agr_recv):
        my = lax.axis_index("i")
        left = (my - 1) % N_DEV
        right = (my + 1) % N_DEV

        a_bf[:, :] = a_ref[:, :].astype(jnp.bfloat16)
        b_bf[:, :] = b_ref[:, :].astype(jnp.bfloat16)

        barrier_sem = pltpu.get_barrier_semaphore()
        for nbr in (left, right):
            pl.semaphore_signal(
                barrier_sem, inc=1,
                device_id=(nbr,), device_id_type=pl.DeviceIdType.MESH,
            )
        pl.semaphore_wait(barrier_sem, 2)

        def partial_f(c):
            return jnp.dot(
                a_bf[pl.ds(c * mc, mc), :], b_bf[:, :],
                preferred_element_type=jnp.float32,
            )

        def partial_r(c):
            return jnp.dot(
                a_bf[pl.ds(half + c * mc, mc), :], b_bf[:, :],
                preferred_element_type=jnp.float32,
            )

        def mk(buf, sends, recvs, s, p, dev):
            rows = pl.ds(p * pm, pm)
            return pltpu.make_async_remote_copy(
                src_ref=buf.at[s, rows, :], dst_ref=buf.at[s + 1, rows, :],
                send_sem=sends.at[s, p], recv_sem=recvs.at[s, p],
                device_id=(dev,), device_id_type=pl.DeviceIdType.MESH,
            )

        d_f, d_r, a_f, a_r = {}, {}, {}, {}
        rsf_buf[0, :, :] = partial_f((my - 1) % N_DEV).astype(jnp.bfloat16)
        rsr_buf[0, :, :] = partial_r((my + 1) % N_DEV).astype(jnp.bfloat16)
        for p in range(PIECES):
            d_f[0, p] = mk(rsf_buf, rsf_send, rsf_recv, 0, p, right)
            d_r[0, p] = mk(rsr_buf, rsr_send, rsr_recv, 0, p, left)
            d_f[0, p].start()
            d_r[0, p].start()

        for s in range(N_DEV - 1):
            pf = partial_f((my - 2 - s) % N_DEV)
            pr = partial_r((my + 2 + s) % N_DEV)
            for p in range(PIECES):
                rows = pl.ds(p * pm, pm)
                d_f[s, p].wait()
                accf = (rsf_buf[s + 1, rows, :].astype(jnp.float32)
                        + pf[p * pm:(p + 1) * pm, :])
                d_r[s, p].wait()
                accr = (rsr_buf[s + 1, rows, :].astype(jnp.float32)
                        + pr[p * pm:(p + 1) * pm, :])
                if s < N_DEV - 2:
                    rsf_buf[s + 1, rows, :] = accf.astype(jnp.bfloat16)
                    d_f[s + 1, p] = mk(rsf_buf, rsf_send, rsf_recv,
                                       s + 1, p, right)
                    d_f[s + 1, p].start()
                    rsr_buf[s + 1, rows, :] = accr.astype(jnp.bfloat16)
                    d_r[s + 1, p] = mk(rsr_buf, rsr_send, rsr_recv,
                                       s + 1, p, left)
                    d_r[s + 1, p].start()
                else:
                    zf = accf * jax.nn.sigmoid(accf)
                    out_ref[pl.ds(my * mc + p * pm, pm), :] = zf
                    agf_buf[0, rows, :] = zf.astype(jnp.bfloat16)
                    a_f[0, p] = mk(agf_buf, agf_send, agf_recv, 0, p, right)
                    a_f[0, p].start()
                    zr = accr * jax.nn.sigmoid(accr)
                    out_ref[pl.ds(half + my * mc + p * pm, pm), :] = zr
                    agr_buf[0, rows, :] = zr.astype(jnp.bfloat16)
                    a_r[0, p] = mk(agr_buf, agr_send, agr_recv, 0, p, left)
                    a_r[0, p].start()

        for h in range(N_DEV - 1):
            cf = (my - h - 1) % N_DEV
            cr = (my + h + 1) % N_DEV
            for p in range(PIECES):
                rows = pl.ds(p * pm, pm)
                a_f[h, p].wait()
                a_r[h, p].wait()
                if h < N_DEV - 2:
                    a_f[h + 1, p] = mk(agf_buf, agf_send, agf_recv,
                                       h + 1, p, right)
                    a_f[h + 1, p].start()
                    a_r[h + 1, p] = mk(agr_buf, agr_send, agr_recv,
                                       h + 1, p, left)
                    a_r[h + 1, p].start()
                out_ref[pl.ds(cf * mc + p * pm, pm), :] = (
                    agf_buf[h + 1, rows, :].astype(jnp.float32))
                out_ref[pl.ds(half + cr * mc + p * pm, pm), :] = (
                    agr_buf[h + 1, rows, :].astype(jnp.float32))

    dma2 = lambda: pltpu.SemaphoreType.DMA((N_DEV - 1, PIECES))
    return pl.pallas_call(
        body,
        out_shape=jax.ShapeDtypeStruct((m, n), jnp.float32),
        in_specs=[
            pl.BlockSpec(memory_space=pltpu.VMEM),
            pl.BlockSpec(memory_space=pltpu.VMEM),
        ],
        out_specs=pl.BlockSpec(memory_space=pltpu.VMEM),
        scratch_shapes=[
            pltpu.VMEM((m, k), jnp.bfloat16),
            pltpu.VMEM((k, n), jnp.bfloat16),
            pltpu.VMEM((N_DEV, mc, n), jnp.bfloat16),
            pltpu.VMEM((N_DEV, mc, n), jnp.bfloat16),
            pltpu.VMEM((N_DEV, mc, n), jnp.bfloat16),
            pltpu.VMEM((N_DEV, mc, n), jnp.bfloat16),
            dma2(), dma2(),
            dma2(), dma2(),
            dma2(), dma2(),
            dma2(), dma2(),
        ],
        compiler_params=pltpu.CompilerParams(
            collective_id=0,
            vmem_limit_bytes=100 * 1024 * 1024,
        ),
    )(A, B)


# device time: 99528 ns/iter; 1.0095x vs baseline; 1.0095x over previous
import jax
import jax.numpy as jnp
from jax import lax
from jax.experimental import pallas as pl
from jax.experimental.pallas import tpu as pltpu

N_DEV = 4
PIECES = 2


def kernel(A, B):
    m, k = A.shape
    _, n = B.shape
    half = m // 2
    mc = half // N_DEV
    pm = mc // PIECES

    def body(a_ref, b_ref, out_ref,
             rsf_buf, rsr_buf, agf_buf, agr_buf,
             rsf_send, rsf_recv, rsr_send, rsr_recv,
             agf_send, agf_recv, agr_send, agr_recv):
        my = lax.axis_index("i")
        left = (my - 1) % N_DEV
        right = (my + 1) % N_DEV

        barrier_sem = pltpu.get_barrier_semaphore()
        for nbr in (left, right):
            pl.semaphore_signal(
                barrier_sem, inc=1,
                device_id=(nbr,), device_id_type=pl.DeviceIdType.MESH,
            )
        pl.semaphore_wait(barrier_sem, 2)

        def partial_f(c):
            return jnp.dot(
                a_ref[pl.ds(c * mc, mc), :], b_ref[:, :],
                preferred_element_type=jnp.float32,
            )

        def partial_r(c):
            return jnp.dot(
                a_ref[pl.ds(half + c * mc, mc), :], b_ref[:, :],
                preferred_element_type=jnp.float32,
            )

        def mk(buf, sends, recvs, s, p, dev):
            rows = pl.ds(p * pm, pm)
            return pltpu.make_async_remote_copy(
                src_ref=buf.at[s, rows, :], dst_ref=buf.at[s + 1, rows, :],
                send_sem=sends.at[s, p], recv_sem=recvs.at[s, p],
                device_id=(dev,), device_id_type=pl.DeviceIdType.MESH,
            )

        d_f, d_r, a_f, a_r = {}, {}, {}, {}
        rsf_buf[0, :, :] = partial_f((my - 1) % N_DEV).astype(jnp.bfloat16)
        rsr_buf[0, :, :] = partial_r((my + 1) % N_DEV).astype(jnp.bfloat16)
        for p in range(PIECES):
            d_f[0, p] = mk(rsf_buf, rsf_send, rsf_recv, 0, p, right)
            d_r[0, p] = mk(rsr_buf, rsr_send, rsr_recv, 0, p, left)
            d_f[0, p].start()
            d_r[0, p].start()

        for s in range(N_DEV - 1):
            pf = partial_f((my - 2 - s) % N_DEV)
            pr = partial_r((my + 2 + s) % N_DEV)
            for p in range(PIECES):
                rows = pl.ds(p * pm, pm)
                d_f[s, p].wait()
                accf = (rsf_buf[s + 1, rows, :].astype(jnp.float32)
                        + pf[p * pm:(p + 1) * pm, :])
                d_r[s, p].wait()
                accr = (rsr_buf[s + 1, rows, :].astype(jnp.float32)
                        + pr[p * pm:(p + 1) * pm, :])
                if s < N_DEV - 2:
                    rsf_buf[s + 1, rows, :] = accf.astype(jnp.bfloat16)
                    d_f[s + 1, p] = mk(rsf_buf, rsf_send, rsf_recv,
                                       s + 1, p, right)
                    d_f[s + 1, p].start()
                    rsr_buf[s + 1, rows, :] = accr.astype(jnp.bfloat16)
                    d_r[s + 1, p] = mk(rsr_buf, rsr_send, rsr_recv,
                                       s + 1, p, left)
                    d_r[s + 1, p].start()
                else:
                    zf = accf * jax.nn.sigmoid(accf)
                    out_ref[pl.ds(my * mc + p * pm, pm), :] = zf
                    agf_buf[0, rows, :] = zf.astype(jnp.bfloat16)
                    a_f[0, p] = mk(agf_buf, agf_send, agf_recv, 0, p, right)
                    a_f[0, p].start()
                    zr = accr * jax.nn.sigmoid(accr)
                    out_ref[pl.ds(half + my * mc + p * pm, pm), :] = zr
                    agr_buf[0, rows, :] = zr.astype(jnp.bfloat16)
                    a_r[0, p] = mk(agr_buf, agr_send, agr_recv, 0, p, left)
                    a_r[0, p].start()

        for h in range(N_DEV - 1):
            cf = (my - h - 1) % N_DEV
            cr = (my + h + 1) % N_DEV
            for p in range(PIECES):
                rows = pl.ds(p * pm, pm)
                a_f[h, p].wait()
                a_r[h, p].wait()
                if h < N_DEV - 2:
                    a_f[h + 1, p] = mk(agf_buf, agf_send, agf_recv,
                                       h + 1, p, right)
                    a_f[h + 1, p].start()
                    a_r[h + 1, p] = mk(agr_buf, agr_send, agr_recv,
                                       h + 1, p, left)
                    a_r[h + 1, p].start()
                out_ref[pl.ds(cf * mc + p * pm, pm), :] = (
                    agf_buf[h + 1, rows, :].astype(jnp.float32))
                out_ref[pl.ds(half + cr * mc + p * pm, pm), :] = (
                    agr_buf[h + 1, rows, :].astype(jnp.float32))

    dma2 = lambda: pltpu.SemaphoreType.DMA((N_DEV - 1, PIECES))
    return pl.pallas_call(
        body,
        out_shape=jax.ShapeDtypeStruct((m, n), jnp.float32),
        in_specs=[
            pl.BlockSpec(memory_space=pltpu.VMEM),
            pl.BlockSpec(memory_space=pltpu.VMEM),
        ],
        out_specs=pl.BlockSpec(memory_space=pltpu.VMEM),
        scratch_shapes=[
            pltpu.VMEM((N_DEV, mc, n), jnp.bfloat16),
            pltpu.VMEM((N_DEV, mc, n), jnp.bfloat16),
            pltpu.VMEM((N_DEV, mc, n), jnp.bfloat16),
            pltpu.VMEM((N_DEV, mc, n), jnp.bfloat16),
            dma2(), dma2(),
            dma2(), dma2(),
            dma2(), dma2(),
            dma2(), dma2(),
        ],
        compiler_params=pltpu.CompilerParams(
            collective_id=0,
            vmem_limit_bytes=100 * 1024 * 1024,
        ),
    )(A, B)
